# baseline (speedup 1.0000x reference)
import math

import jax
import jax.numpy as jnp
import numpy as np
from jax import lax
from jax.experimental import pallas as pl
from jax.experimental.pallas import tpu as pltpu
from jax.experimental.pallas import tpu_sc as plsc

_T = 64
_N = 4096
_C = 128

_SC_CORES = 2
_SC_SUBCORES = 16
_NW = _SC_CORES * _SC_SUBCORES
_FLAT = _N * _C
_PER_W = _FLAT // _NW
_CH = 512
_NCHUNK = _PER_W // _CH
_L = 16


def _sg_endpoint_weights() -> np.ndarray:
    K, p, m, dt = _T, 2, 1, 0.02
    try:
        x = jnp.arange(-K + 1, 1, dtype=jnp.float32) * float(dt)
        A = jnp.stack([x**j for j in range(p + 1)], axis=1)
        ATA_pinv = jnp.linalg.pinv(A.T @ A)
        e_m = jnp.zeros(p + 1, dtype=jnp.float32).at[m].set(1.0)
        w = (e_m @ ATA_pinv @ A.T) * float(math.factorial(m))
        return np.asarray(w, dtype=np.float32)
    except Exception:
        xn = np.arange(-K + 1, 1, dtype=np.float64) * dt
        An = np.stack([xn**j for j in range(p + 1)], axis=1)
        en = np.zeros(p + 1)
        en[m] = 1.0
        wn = en @ np.linalg.pinv(An.T @ An) @ An.T
        return (wn * float(math.factorial(m))).astype(np.float32)


_W = _sg_endpoint_weights()
_WF = [float(v) for v in _W]


def _sc_body(x_hbm, o_hbm, buf, out_v):
    c = lax.axis_index("c")
    s = lax.axis_index("s")
    wid = s * _SC_CORES + c
    base = wid * _PER_W

    def do_chunk(k, carry):
        col = base + k * _CH
        pltpu.sync_copy(x_hbm.at[:, pl.ds(col, _CH)], buf)

        def inner(i, carry2):
            acc = _WF[0] * buf[0, pl.ds(i * _L, _L)]
            for t in range(1, _T):
                acc = acc + _WF[t] * buf[t, pl.ds(i * _L, _L)]
            out_v[pl.ds(k * _CH + i * _L, _L)] = acc
            return carry2

        lax.fori_loop(0, _CH // _L, inner, 0)
        return carry

    lax.fori_loop(0, _NCHUNK, do_chunk, 0)
    pltpu.sync_copy(out_v, o_hbm.at[pl.ds(base, _PER_W)])


def _sc_call(x2):
    mesh = plsc.VectorSubcoreMesh(core_axis_name="c", subcore_axis_name="s")
    return pl.kernel(
        _sc_body,
        out_type=jax.ShapeDtypeStruct((_FLAT,), jnp.float32),
        mesh=mesh,
        scratch_types=[
            pltpu.VMEM((_T, _CH), jnp.float32),
            pltpu.VMEM((_PER_W,), jnp.float32),
        ],
    )(x2)


def kernel(x):
    out_flat = _sc_call(x.reshape(_T, _FLAT))
    return out_flat.reshape(_N, _C)


# speedup vs baseline: 2.4572x; 2.4572x over previous
import math

import jax
import jax.numpy as jnp
import numpy as np
from jax import lax
from jax.experimental import pallas as pl
from jax.experimental.pallas import tpu as pltpu
from jax.experimental.pallas import tpu_sc as plsc

_T = 64
_N = 4096
_C = 128

_SC_CORES = 2
_SC_SUBCORES = 16
_NW = _SC_CORES * _SC_SUBCORES
_EPW = _N // _NW
_EB = 4
_NCH = _EPW // _EB
_L = 16
_GPC = _EB * _C // _L


def _sg_endpoint_weights() -> np.ndarray:
    K, p, m, dt = _T, 2, 1, 0.02
    try:
        x = jnp.arange(-K + 1, 1, dtype=jnp.float32) * float(dt)
        A = jnp.stack([x**j for j in range(p + 1)], axis=1)
        ATA_pinv = jnp.linalg.pinv(A.T @ A)
        e_m = jnp.zeros(p + 1, dtype=jnp.float32).at[m].set(1.0)
        w = (e_m @ ATA_pinv @ A.T) * float(math.factorial(m))
        return np.asarray(w, dtype=np.float32)
    except Exception:
        xn = np.arange(-K + 1, 1, dtype=np.float64) * dt
        An = np.stack([xn**j for j in range(p + 1)], axis=1)
        en = np.zeros(p + 1)
        en[m] = 1.0
        wn = en @ np.linalg.pinv(An.T @ An) @ An.T
        return (wn * float(math.factorial(m))).astype(np.float32)


_W = _sg_endpoint_weights()
_WF = [float(v) for v in _W]


def _sc_body(x_hbm, o_hbm, buf, out_v, sem0, sem1):
    c = lax.axis_index("c")
    s = lax.axis_index("s")
    wid = s * _SC_CORES + c
    e0 = wid * _EPW
    sems = (sem0, sem1)

    def fire(k, b):
        pltpu.async_copy(
            x_hbm.at[:, pl.ds(e0 + k * _EB, _EB), :], buf.at[b], sems[b]
        )

    fire(0, 0)

    def pair(j, carry):
        k0 = j * 2
        for b in range(2):
            k = k0 + b

            @pl.when(k + 1 < _NCH)
            def _fire_next():
                fire(k + 1, 1 - b)

            pltpu.make_async_copy(
                x_hbm.at[:, pl.ds(e0, _EB), :], buf.at[b], sems[b]
            ).wait()

            def group(i, carry2):
                e = i // (_C // _L)
                off = (i % (_C // _L)) * _L
                acc = _WF[0] * buf[b, 0, e, pl.ds(off, _L)]
                for t in range(1, _T):
                    acc = acc + _WF[t] * buf[b, t, e, pl.ds(off, _L)]
                out_v[k * _EB + e, pl.ds(off, _L)] = acc
                return carry2

            lax.fori_loop(0, _GPC, group, 0)
        return carry

    lax.fori_loop(0, _NCH // 2, pair, 0)
    pltpu.sync_copy(out_v, o_hbm.at[pl.ds(e0, _EPW), :])


def _sc_call(x):
    mesh = plsc.VectorSubcoreMesh(core_axis_name="c", subcore_axis_name="s")
    return pl.kernel(
        _sc_body,
        out_type=jax.ShapeDtypeStruct((_N, _C), jnp.float32),
        mesh=mesh,
        scratch_types=[
            pltpu.VMEM((2, _T, _EB, _C), jnp.float32),
            pltpu.VMEM((_EPW, _C), jnp.float32),
            pltpu.SemaphoreType.DMA,
            pltpu.SemaphoreType.DMA,
        ],
    )(x)


def kernel(x):
    return _sc_call(x)
